# Initial kernel scaffold; baseline (speedup 1.0000x reference)
import jax
import jax.numpy as jnp
from jax.experimental import pallas as pl


def kernel(x, freqs_cis, wqkv, wo, input_pos):
    raise NotImplementedError("write your pallas kernel here")


# speedup vs baseline: 1.1695x; 1.1695x over previous
import functools
import math

import jax
import jax.numpy as jnp
from jax.experimental import pallas as pl
from jax.experimental.pallas import tpu as pltpu

S = 4096
DIM = 2048
NH, HD = 16, 128
BS = 8
TB = S // BS
SINK_B = 4
WIN_B = 4
CUR_BLOCK = TB - 1
WIN_START = CUR_BLOCK - WIN_B + 1
MB = 512 // BS
KL = SINK_B + WIN_B + MB
KSEL = KL * BS
SCALE = 1.0 / math.sqrt(HD)
NEG = -1e30


def _mm_kernel(a_ref, b_ref, o_ref):
    o_ref[:, :] = jax.lax.dot_general(
        a_ref[:, :], b_ref[:, :], (((1,), (0,)), ((), ())),
        preferred_element_type=jnp.float32)


def _matmul(a, b, bm, bn):
    m, k = a.shape
    k2, n = b.shape
    return pl.pallas_call(
        _mm_kernel,
        grid=(m // bm, n // bn),
        in_specs=[pl.BlockSpec((bm, k), lambda i, j: (i, 0)),
                  pl.BlockSpec((k, bn), lambda i, j: (0, j))],
        out_specs=pl.BlockSpec((bm, bn), lambda i, j: (i, j)),
        out_shape=jax.ShapeDtypeStruct((m, n), jnp.float32),
    )(a, b)


def _rope_kernel(q_ref, k_ref, cos_ref, sin_ref, qr_ref, kr_ref, kb_ref):
    c = cos_ref[:, :, :]
    s = sin_ref[:, :, :]
    q = q_ref[:, :, :]
    ql, qh = q[:, :, :64], q[:, :, 64:]
    qr_ref[:, :, :64] = ql * c - qh * s
    qr_ref[:, :, 64:] = qh * c + ql * s
    k = k_ref[:, :, :]
    kl_, kh = k[:, :, :64], k[:, :, 64:]
    krl = kl_ * c - kh * s
    krh = kh * c + kl_ * s
    kr_ref[:, :, :64] = krl
    kr_ref[:, :, 64:] = krh
    kr = jnp.concatenate([krl, krh], axis=-1)
    bs = kr.shape[0]
    kb_ref[:, :, :] = jnp.mean(kr.reshape(bs // BS, BS, NH, HD), axis=1)


def _rope(q3, k3, cos3, sin3, bs):
    return pl.pallas_call(
        _rope_kernel,
        grid=(S // bs,),
        in_specs=[pl.BlockSpec((bs, NH, HD), lambda i: (i, 0, 0)),
                  pl.BlockSpec((bs, NH, HD), lambda i: (i, 0, 0)),
                  pl.BlockSpec((bs, 1, 64), lambda i: (i, 0, 0)),
                  pl.BlockSpec((bs, 1, 64), lambda i: (i, 0, 0))],
        out_specs=[pl.BlockSpec((bs, NH, HD), lambda i: (i, 0, 0)),
                   pl.BlockSpec((bs, NH, HD), lambda i: (i, 0, 0)),
                   pl.BlockSpec((bs // BS, NH, HD), lambda i: (i, 0, 0))],
        out_shape=[jax.ShapeDtypeStruct((S, NH, HD), jnp.float32),
                   jax.ShapeDtypeStruct((S, NH, HD), jnp.float32),
                   jax.ShapeDtypeStruct((TB, NH, HD), jnp.float32)],
    )(q3, k3, cos3, sin3)


def _route_kernel(kb_ref, ql_ref, top_ref):
    kb = kb_ref[:, :, :]
    ql = ql_ref[:, :, :]
    scores = jnp.sum(kb * ql, axis=-1)
    rid = jax.lax.broadcasted_iota(jnp.int32, (TB, NH), 0)
    excl = (rid < SINK_B) | (rid >= WIN_START)
    scores = jnp.where(excl, NEG, scores)

    def body(j, sc):
        idx = jnp.argmax(sc, axis=0).astype(jnp.int32)
        top_ref[pl.ds(j, 1), :] = idx[None, :]
        hit = rid == idx[None, :]
        return jnp.where(hit, -jnp.inf, sc)

    jax.lax.fori_loop(0, MB, body, scores)


def _route(k_blk, q_last3):
    return pl.pallas_call(
        _route_kernel,
        grid=(1,),
        in_specs=[pl.BlockSpec((TB, NH, HD), lambda i: (0, 0, 0)),
                  pl.BlockSpec((1, NH, HD), lambda i: (0, 0, 0))],
        out_specs=pl.BlockSpec((MB, NH), lambda i: (0, 0)),
        out_shape=jax.ShapeDtypeStruct((MB, NH), jnp.int32),
    )(k_blk, q_last3)


def _attn_kernel(bi_ref, pos_ref, q_ref, k_ref, v_ref, o_ref, ks_ref, vs_ref):
    qt = pl.program_id(1)

    @pl.when(qt == 0)
    def _gather():
        def body(j, _):
            blk = bi_ref[0, 0, j]
            ks_ref[pl.ds(j * BS, BS), :] = k_ref[pl.ds(blk * BS, BS), :]
            vs_ref[pl.ds(j * BS, BS), :] = v_ref[pl.ds(blk * BS, BS), :]
            return 0
        jax.lax.fori_loop(0, KL, body, 0)

    q = q_ref[:, :]
    tq = q.shape[0]
    att = jax.lax.dot_general(
        q, ks_ref[:, :], (((1,), (1,)), ((), ())),
        preferred_element_type=jnp.float32) * SCALE
    qpos = (qt * tq + jax.lax.broadcasted_iota(jnp.int32, (tq, 1), 0)
            ).astype(jnp.float32)
    allow = pos_ref[0, :, :] <= qpos
    att = jnp.where(allow, att, NEG)
    m = jnp.max(att, axis=1, keepdims=True)
    e = jnp.exp(att - m)
    denom = jnp.sum(e, axis=1, keepdims=True)
    y = jax.lax.dot_general(
        e, vs_ref[:, :], (((1,), (0,)), ((), ())),
        preferred_element_type=jnp.float32)
    o_ref[:, :] = y / denom


def _attention(block_index, pos3, q2, k2, v2, tq):
    return pl.pallas_call(
        _attn_kernel,
        grid=(NH, S // tq),
        in_specs=[
            pl.BlockSpec((1, 1, KL), lambda h, i: (h, 0, 0),
                         memory_space=pltpu.SMEM),
            pl.BlockSpec((1, 1, KSEL), lambda h, i: (h, 0, 0)),
            pl.BlockSpec((tq, HD), lambda h, i: (i, h)),
            pl.BlockSpec((S, HD), lambda h, i: (0, h)),
            pl.BlockSpec((S, HD), lambda h, i: (0, h)),
        ],
        out_specs=pl.BlockSpec((tq, HD), lambda h, i: (i, h)),
        out_shape=jax.ShapeDtypeStruct((S, NH * HD), jnp.float32),
        scratch_shapes=[pltpu.VMEM((KSEL, HD), jnp.float32),
                        pltpu.VMEM((KSEL, HD), jnp.float32)],
    )(block_index.reshape(NH, 1, KL), pos3, q2, k2, v2)


def _permute_qk_cols(w):
    return (w.reshape(DIM, NH, HD // 2, 2)
             .transpose(0, 1, 3, 2)
             .reshape(DIM, NH * HD))


def kernel(x, freqs_cis, wqkv, wo, input_pos):
    x2 = x[0]
    wq = _permute_qk_cols(wqkv[:, :NH * HD])
    wk = _permute_qk_cols(wqkv[:, NH * HD:2 * NH * HD])
    wv = wqkv[:, 2 * NH * HD:]
    wqkv_p = jnp.concatenate([wq, wk, wv], axis=1)

    cos3 = freqs_cis[:, :, 0].reshape(S, 1, HD // 2)
    sin3 = freqs_cis[:, :, 1].reshape(S, 1, HD // 2)

    qkv = _matmul(x2, wqkv_p, 256, 512)
    q3 = qkv[:, :NH * HD].reshape(S, NH, HD)
    k3 = qkv[:, NH * HD:2 * NH * HD].reshape(S, NH, HD)
    v2 = qkv[:, 2 * NH * HD:]

    q_rot, k_rot, k_blk = _rope(q3, k3, cos3, sin3, 256)

    q_last3 = q_rot[S - 1].reshape(1, NH, HD)
    top = _route(k_blk, q_last3)

    fixed = jnp.concatenate([
        jnp.arange(SINK_B, dtype=jnp.int32),
        jnp.arange(WIN_START, CUR_BLOCK + 1, dtype=jnp.int32)])
    block_index = jnp.concatenate(
        [jnp.broadcast_to(fixed[None, :], (NH, SINK_B + WIN_B)),
         top.T], axis=1)
    pos3 = (block_index[:, :, None] * BS
            + jnp.arange(BS, dtype=jnp.int32)[None, None, :]
            ).reshape(NH, 1, KSEL).astype(jnp.float32)

    q2 = q_rot.reshape(S, NH * HD)
    k2 = k_rot.reshape(S, NH * HD)
    y2 = _attention(block_index, pos3, q2, k2, v2, 256)

    out = _matmul(y2, wo, 512, 512)
    return out.reshape(1, S, DIM)


# speedup vs baseline: 1.6159x; 1.3817x over previous
import functools
import math

import jax
import jax.numpy as jnp
from jax.experimental import pallas as pl
from jax.experimental.pallas import tpu as pltpu

S = 4096
DIM = 2048
NH, HD = 16, 128
BS = 8
TB = S // BS
SINK_B = 4
WIN_B = 4
CUR_BLOCK = TB - 1
WIN_START = CUR_BLOCK - WIN_B + 1
MB = 512 // BS
KL = SINK_B + WIN_B + MB
KSEL = KL * BS
SCALE = 1.0 / math.sqrt(HD)
NEG = -1e30


def _mm_kernel(a_ref, b_ref, o_ref):
    o_ref[:, :] = jax.lax.dot_general(
        a_ref[:, :], b_ref[:, :], (((1,), (0,)), ((), ())),
        preferred_element_type=jnp.float32).astype(o_ref.dtype)


def _matmul(a, b, bm, bn, out_dtype):
    m, k = a.shape
    k2, n = b.shape
    return pl.pallas_call(
        _mm_kernel,
        grid=(n // bn, m // bm),
        in_specs=[pl.BlockSpec((bm, k), lambda j, i: (i, 0)),
                  pl.BlockSpec((k, bn), lambda j, i: (0, j))],
        out_specs=pl.BlockSpec((bm, bn), lambda j, i: (i, j)),
        out_shape=jax.ShapeDtypeStruct((m, n), out_dtype),
    )(a, b)


def _rope_kernel(q_ref, k_ref, cos_ref, sin_ref, qr_ref, kr_ref, kb_ref):
    c = cos_ref[:, :, :]
    s = sin_ref[:, :, :]
    q = q_ref[:, :, :].astype(jnp.float32)
    ql, qh = q[:, :, :64], q[:, :, 64:]
    qr_ref[:, :, :64] = (ql * c - qh * s).astype(qr_ref.dtype)
    qr_ref[:, :, 64:] = (qh * c + ql * s).astype(qr_ref.dtype)
    k = k_ref[:, :, :].astype(jnp.float32)
    kl_, kh = k[:, :, :64], k[:, :, 64:]
    krl = kl_ * c - kh * s
    krh = kh * c + kl_ * s
    kr_ref[:, :, :64] = krl.astype(kr_ref.dtype)
    kr_ref[:, :, 64:] = krh.astype(kr_ref.dtype)
    kr = jnp.concatenate([krl, krh], axis=-1)
    bs = kr.shape[0]
    kb_ref[:, :, :] = jnp.mean(kr.reshape(bs // BS, BS, NH, HD), axis=1)


def _rope(q3, k3, cos3, sin3, bs):
    return pl.pallas_call(
        _rope_kernel,
        grid=(S // bs,),
        in_specs=[pl.BlockSpec((bs, NH, HD), lambda i: (i, 0, 0)),
                  pl.BlockSpec((bs, NH, HD), lambda i: (i, 0, 0)),
                  pl.BlockSpec((bs, 1, 64), lambda i: (i, 0, 0)),
                  pl.BlockSpec((bs, 1, 64), lambda i: (i, 0, 0))],
        out_specs=[pl.BlockSpec((bs, NH, HD), lambda i: (i, 0, 0)),
                   pl.BlockSpec((bs, NH, HD), lambda i: (i, 0, 0)),
                   pl.BlockSpec((bs // BS, NH, HD), lambda i: (i, 0, 0))],
        out_shape=[jax.ShapeDtypeStruct((S, NH, HD), jnp.bfloat16),
                   jax.ShapeDtypeStruct((S, NH, HD), jnp.bfloat16),
                   jax.ShapeDtypeStruct((TB, NH, HD), jnp.float32)],
    )(q3, k3, cos3, sin3)


def _route_kernel(kb_ref, ql_ref, cos_ref, sin_ref, top_ref):
    kb = kb_ref[:, :, :]
    qraw = ql_ref[:, :, :]
    c = cos_ref[:, :, :]
    s = sin_ref[:, :, :]
    qlo, qhi = qraw[:, :, :64], qraw[:, :, 64:]
    ql = jnp.concatenate([qlo * c - qhi * s, qhi * c + qlo * s], axis=-1)
    kb16 = kb.astype(jnp.bfloat16).astype(jnp.float32)
    ql16 = ql.astype(jnp.bfloat16).astype(jnp.float32)
    scores = jnp.sum(kb16 * ql16, axis=-1)
    rid = jax.lax.broadcasted_iota(jnp.int32, (TB, NH), 0)
    excl = (rid < SINK_B) | (rid >= WIN_START)
    scores = jnp.where(excl, NEG, scores)

    def body(j, sc):
        idx = jnp.argmax(sc, axis=0).astype(jnp.int32)
        top_ref[pl.ds(j, 1), :] = idx[None, :]
        hit = rid == idx[None, :]
        return jnp.where(hit, -jnp.inf, sc)

    jax.lax.fori_loop(0, MB, body, scores)


def _route(k_blk, q_last3, cos_last, sin_last):
    return pl.pallas_call(
        _route_kernel,
        grid=(1,),
        in_specs=[pl.BlockSpec((TB, NH, HD), lambda i: (0, 0, 0)),
                  pl.BlockSpec((1, NH, HD), lambda i: (0, 0, 0)),
                  pl.BlockSpec((1, 1, 64), lambda i: (0, 0, 0)),
                  pl.BlockSpec((1, 1, 64), lambda i: (0, 0, 0))],
        out_specs=pl.BlockSpec((MB, NH), lambda i: (0, 0)),
        out_shape=jax.ShapeDtypeStruct((MB, NH), jnp.int32),
    )(k_blk, q_last3, cos_last, sin_last)


def _attn_kernel(bi_ref, pos_ref, q_ref, k_ref, v_ref, o_ref, ks_ref, vs_ref):
    qt = pl.program_id(1)

    @pl.when(qt == 0)
    def _gather():
        def body(j, _):
            blk = bi_ref[0, 0, j]
            ks_ref[pl.ds(j * BS, BS), :] = k_ref[pl.ds(blk * BS, BS), :]
            vs_ref[pl.ds(j * BS, BS), :] = v_ref[pl.ds(blk * BS, BS), :]
            return 0
        jax.lax.fori_loop(0, KL, body, 0)

    q = q_ref[:, :]
    tq = q.shape[0]
    att = jax.lax.dot_general(
        q, ks_ref[:, :], (((1,), (1,)), ((), ())),
        preferred_element_type=jnp.float32) * SCALE
    qpos = (qt * tq + jax.lax.broadcasted_iota(jnp.int32, (tq, 1), 0)
            ).astype(jnp.float32)
    allow = pos_ref[0, :, :] <= qpos
    att = jnp.where(allow, att, NEG)
    m = jnp.max(att, axis=1, keepdims=True)
    e = jnp.exp(att - m)
    denom = jnp.sum(e, axis=1, keepdims=True)
    y = jax.lax.dot_general(
        e.astype(jnp.bfloat16), vs_ref[:, :], (((1,), (0,)), ((), ())),
        preferred_element_type=jnp.float32)
    o_ref[:, :] = (y / denom).astype(o_ref.dtype)


def _attention(block_index, pos3, q2, k2, v2, tq):
    return pl.pallas_call(
        _attn_kernel,
        grid=(NH, S // tq),
        in_specs=[
            pl.BlockSpec((1, 1, KL), lambda h, i: (h, 0, 0),
                         memory_space=pltpu.SMEM),
            pl.BlockSpec((1, 1, KSEL), lambda h, i: (h, 0, 0)),
            pl.BlockSpec((tq, HD), lambda h, i: (i, h)),
            pl.BlockSpec((S, HD), lambda h, i: (0, h)),
            pl.BlockSpec((S, HD), lambda h, i: (0, h)),
        ],
        out_specs=pl.BlockSpec((tq, HD), lambda h, i: (i, h)),
        out_shape=jax.ShapeDtypeStruct((S, NH * HD), jnp.bfloat16),
        scratch_shapes=[pltpu.VMEM((KSEL, HD), jnp.bfloat16),
                        pltpu.VMEM((KSEL, HD), jnp.bfloat16)],
    )(block_index.reshape(NH, 1, KL), pos3, q2, k2, v2)


def _permute_qk_cols(w):
    return (w.reshape(DIM, NH, HD // 2, 2)
             .transpose(0, 1, 3, 2)
             .reshape(DIM, NH * HD))


def kernel(x, freqs_cis, wqkv, wo, input_pos):
    x2f = x[0]
    x2b = x2f.astype(jnp.bfloat16)
    wq = _permute_qk_cols(wqkv[:, :NH * HD])
    wk = _permute_qk_cols(wqkv[:, NH * HD:2 * NH * HD])
    wv = wqkv[:, 2 * NH * HD:]
    wqv_b = jnp.concatenate([wq, wv], axis=1).astype(jnp.bfloat16)

    cos3 = freqs_cis[:, :, 0].reshape(S, 1, HD // 2)
    sin3 = freqs_cis[:, :, 1].reshape(S, 1, HD // 2)

    qv = _matmul(x2b, wqv_b, 512, 2048, jnp.bfloat16)
    kf = _matmul(x2f, wk, 512, 2048, jnp.float32)
    qlast8 = _matmul(x2f[S - 8:], wq, 8, 2048, jnp.float32)

    q3 = qv[:, :NH * HD].reshape(S, NH, HD)
    v2 = qv[:, NH * HD:]
    k3 = kf.reshape(S, NH, HD)

    q_rot, k_rot, k_blk = _rope(q3, k3, cos3, sin3, 256)

    q_last3 = qlast8[7].reshape(1, NH, HD)
    cos_last = freqs_cis[S - 1, :, 0].reshape(1, 1, HD // 2)
    sin_last = freqs_cis[S - 1, :, 1].reshape(1, 1, HD // 2)
    top = _route(k_blk, q_last3, cos_last, sin_last)

    fixed = jnp.concatenate([
        jnp.arange(SINK_B, dtype=jnp.int32),
        jnp.arange(WIN_START, CUR_BLOCK + 1, dtype=jnp.int32)])
    block_index = jnp.concatenate(
        [jnp.broadcast_to(fixed[None, :], (NH, SINK_B + WIN_B)),
         top.T], axis=1)
    pos3 = (block_index[:, :, None] * BS
            + jnp.arange(BS, dtype=jnp.int32)[None, None, :]
            ).reshape(NH, 1, KSEL).astype(jnp.float32)

    q2 = q_rot.reshape(S, NH * HD)
    k2 = k_rot.reshape(S, NH * HD)
    y2 = _attention(block_index, pos3, q2, k2, v2, 256)

    out = _matmul(y2, wo.astype(jnp.bfloat16), 512, 2048, jnp.float32)
    return out.reshape(1, S, DIM)


# speedup vs baseline: 2.1779x; 1.3478x over previous
import math

import jax
import jax.numpy as jnp
from jax.experimental import pallas as pl
from jax.experimental.pallas import tpu as pltpu

S = 4096
DIM = 2048
NH, HD = 16, 128
HH = HD // 2
BS = 8
TB = S // BS
SINK_B = 4
WIN_B = 4
CUR_BLOCK = TB - 1
WIN_START = CUR_BLOCK - WIN_B + 1
MB = 512 // BS
KL = SINK_B + WIN_B + MB
KSEL = KL * BS
SCALE = 1.0 / math.sqrt(HD)
NEG = -1e30
KOFF = NH * HD
VOFF = 2 * NH * HD


def _qkvrope_kernel(x_ref, w_ref, cos_ref, sin_ref,
                    qr_ref, kr_ref, v_ref, kb_ref, ql_ref):
    i = pl.program_id(0)
    acc = jax.lax.dot_general(
        x_ref[:, :], w_ref[:, :], (((1,), (0,)), ((), ())),
        preferred_element_type=jnp.float32)
    c = cos_ref[:, :]
    s = sin_ref[:, :]
    bm = acc.shape[0]
    for h in range(NH):
        qh = acc[:, h * HD:(h + 1) * HD]
        lo, hi = qh[:, :HH], qh[:, HH:]
        qr_ref[:, h * HD:h * HD + HH] = (lo * c - hi * s).astype(jnp.bfloat16)
        qr_ref[:, h * HD + HH:(h + 1) * HD] = (hi * c + lo * s).astype(jnp.bfloat16)
        kh = acc[:, KOFF + h * HD:KOFF + (h + 1) * HD]
        klo, khi = kh[:, :HH], kh[:, HH:]
        krl = klo * c - khi * s
        krh = khi * c + klo * s
        kr_ref[:, h * HD:h * HD + HH] = krl.astype(jnp.bfloat16)
        kr_ref[:, h * HD + HH:(h + 1) * HD] = krh.astype(jnp.bfloat16)
        kroth = jnp.concatenate([krl, krh], axis=1)
        kb_ref[:, h * HD:(h + 1) * HD] = jnp.mean(
            kroth.reshape(bm // BS, BS, HD), axis=1)
    v_ref[:, :] = acc[:, VOFF:].astype(jnp.bfloat16)

    @pl.when(i == pl.num_programs(0) - 1)
    def _():
        ql_ref[:, :] = acc[bm - 8:, :KOFF]


def _qkv_rope(x2b, w_all, cos2, sin2, bm):
    return pl.pallas_call(
        _qkvrope_kernel,
        grid=(S // bm,),
        in_specs=[pl.BlockSpec((bm, DIM), lambda i: (i, 0)),
                  pl.BlockSpec((DIM, 3 * NH * HD), lambda i: (0, 0)),
                  pl.BlockSpec((bm, HH), lambda i: (i, 0)),
                  pl.BlockSpec((bm, HH), lambda i: (i, 0))],
        out_specs=[pl.BlockSpec((bm, NH * HD), lambda i: (i, 0)),
                   pl.BlockSpec((bm, NH * HD), lambda i: (i, 0)),
                   pl.BlockSpec((bm, NH * HD), lambda i: (i, 0)),
                   pl.BlockSpec((bm // BS, NH * HD), lambda i: (i, 0)),
                   pl.BlockSpec((8, NH * HD), lambda i: (0, 0))],
        out_shape=[jax.ShapeDtypeStruct((S, NH * HD), jnp.bfloat16),
                   jax.ShapeDtypeStruct((S, NH * HD), jnp.bfloat16),
                   jax.ShapeDtypeStruct((S, NH * HD), jnp.bfloat16),
                   jax.ShapeDtypeStruct((TB, NH * HD), jnp.float32),
                   jax.ShapeDtypeStruct((8, NH * HD), jnp.float32)],
    )(x2b, w_all, cos2, sin2)


def _route_kernel(kb_ref, ql_ref, cos_ref, sin_ref, top_ref):
    c = cos_ref[:, :]
    s = sin_ref[:, :]
    qraw = ql_ref[7:8, :]
    cols = []
    for h in range(NH):
        qh = qraw[:, h * HD:(h + 1) * HD]
        lo, hi = qh[:, :HH], qh[:, HH:]
        qlh = jnp.concatenate([lo * c - hi * s, hi * c + lo * s], axis=1)
        kb16 = kb_ref[:, h * HD:(h + 1) * HD].astype(jnp.bfloat16)
        ql16 = qlh.astype(jnp.bfloat16).astype(jnp.float32)
        cols.append(jnp.sum(kb16.astype(jnp.float32) * ql16,
                            axis=1, keepdims=True))
    scores = jnp.concatenate(cols, axis=1)
    rid = jax.lax.broadcasted_iota(jnp.int32, (TB, NH), 0)
    excl = (rid < SINK_B) | (rid >= WIN_START)
    scores = jnp.where(excl, NEG, scores)

    def body(j, sc):
        idx = jnp.argmax(sc, axis=0).astype(jnp.int32)
        top_ref[pl.ds(j, 1), :] = idx[None, :]
        hit = rid == idx[None, :]
        return jnp.where(hit, -jnp.inf, sc)

    jax.lax.fori_loop(0, MB, body, scores)


def _route(kb2, ql8, cos_last, sin_last):
    return pl.pallas_call(
        _route_kernel,
        grid=(1,),
        in_specs=[pl.BlockSpec((TB, NH * HD), lambda i: (0, 0)),
                  pl.BlockSpec((8, NH * HD), lambda i: (0, 0)),
                  pl.BlockSpec((1, HH), lambda i: (0, 0)),
                  pl.BlockSpec((1, HH), lambda i: (0, 0))],
        out_specs=pl.BlockSpec((MB, NH), lambda i: (0, 0)),
        out_shape=jax.ShapeDtypeStruct((MB, NH), jnp.int32),
    )(kb2, ql8, cos_last, sin_last)


def _attn_kernel(bi_ref, pos_ref, q_ref, k_ref, v_ref, o_ref, ks_ref, vs_ref):
    qt = pl.program_id(1)

    @pl.when(qt == 0)
    def _gather():
        def body(j, _):
            blk = bi_ref[0, 0, j]
            ks_ref[pl.ds(j * BS, BS), :] = k_ref[pl.ds(blk * BS, BS), :]
            vs_ref[pl.ds(j * BS, BS), :] = v_ref[pl.ds(blk * BS, BS), :]
            return 0
        jax.lax.fori_loop(0, KL, body, 0)

    q = q_ref[:, :]
    tq = q.shape[0]
    att = jax.lax.dot_general(
        q, ks_ref[:, :], (((1,), (1,)), ((), ())),
        preferred_element_type=jnp.float32) * SCALE
    qpos = (qt * tq + jax.lax.broadcasted_iota(jnp.int32, (tq, 1), 0)
            ).astype(jnp.float32)
    allow = pos_ref[0, :, :] <= qpos
    att = jnp.where(allow, att, NEG)
    m = jnp.max(att, axis=1, keepdims=True)
    e = jnp.exp(att - m)
    denom = jnp.sum(e, axis=1, keepdims=True)
    y = jax.lax.dot_general(
        e.astype(jnp.bfloat16), vs_ref[:, :], (((1,), (0,)), ((), ())),
        preferred_element_type=jnp.float32)
    o_ref[:, :] = (y / denom).astype(o_ref.dtype)


def _attention(block_index, pos3, q2, k2, v2, tq):
    return pl.pallas_call(
        _attn_kernel,
        grid=(NH, S // tq),
        in_specs=[
            pl.BlockSpec((1, 1, KL), lambda h, i: (h, 0, 0),
                         memory_space=pltpu.SMEM),
            pl.BlockSpec((1, 1, KSEL), lambda h, i: (h, 0, 0)),
            pl.BlockSpec((tq, HD), lambda h, i: (i, h)),
            pl.BlockSpec((S, HD), lambda h, i: (0, h)),
            pl.BlockSpec((S, HD), lambda h, i: (0, h)),
        ],
        out_specs=pl.BlockSpec((tq, HD), lambda h, i: (i, h)),
        out_shape=jax.ShapeDtypeStruct((S, NH * HD), jnp.bfloat16),
        scratch_shapes=[pltpu.VMEM((KSEL, HD), jnp.bfloat16),
                        pltpu.VMEM((KSEL, HD), jnp.bfloat16)],
    )(block_index.reshape(NH, 1, KL), pos3, q2, k2, v2)


def _mm_kernel(a_ref, b_ref, o_ref):
    o_ref[:, :] = jax.lax.dot_general(
        a_ref[:, :], b_ref[:, :].astype(jnp.bfloat16),
        (((1,), (0,)), ((), ())),
        preferred_element_type=jnp.float32)


def _out_proj(a, b, bm):
    m, k = a.shape
    k2, n = b.shape
    return pl.pallas_call(
        _mm_kernel,
        grid=(m // bm,),
        in_specs=[pl.BlockSpec((bm, k), lambda i: (i, 0)),
                  pl.BlockSpec((k, n), lambda i: (0, 0))],
        out_specs=pl.BlockSpec((bm, n), lambda i: (i, 0)),
        out_shape=jax.ShapeDtypeStruct((m, n), jnp.float32),
    )(a, b)


def _permute_qk_cols(w):
    return (w.reshape(DIM, NH, HH, 2)
             .transpose(0, 1, 3, 2)
             .reshape(DIM, NH * HD))


def kernel(x, freqs_cis, wqkv, wo, input_pos):
    x2b = x[0].astype(jnp.bfloat16)
    wq = _permute_qk_cols(wqkv[:, :KOFF])
    wk = _permute_qk_cols(wqkv[:, KOFF:VOFF])
    w_all = jnp.concatenate([wq, wk, wqkv[:, VOFF:]],
                            axis=1).astype(jnp.bfloat16)

    cos2 = freqs_cis[:, :, 0]
    sin2 = freqs_cis[:, :, 1]

    q_rot, k_rot, v2, kb2, ql8 = _qkv_rope(x2b, w_all, cos2, sin2, 256)

    cos_last = freqs_cis[S - 1, :, 0].reshape(1, HH)
    sin_last = freqs_cis[S - 1, :, 1].reshape(1, HH)
    top = _route(kb2, ql8, cos_last, sin_last)

    fixed = jnp.concatenate([
        jnp.arange(SINK_B, dtype=jnp.int32),
        jnp.arange(WIN_START, CUR_BLOCK + 1, dtype=jnp.int32)])
    block_index = jnp.concatenate(
        [jnp.broadcast_to(fixed[None, :], (NH, SINK_B + WIN_B)),
         top.T], axis=1)
    pos3 = (block_index[:, :, None] * BS
            + jnp.arange(BS, dtype=jnp.int32)[None, None, :]
            ).reshape(NH, 1, KSEL).astype(jnp.float32)

    y2 = _attention(block_index, pos3, q_rot, k_rot, v2, 256)

    out = _out_proj(y2, wo, 512)
    return out.reshape(1, S, DIM)


# speedup vs baseline: 2.7877x; 1.2800x over previous
import math

import jax
import jax.numpy as jnp
from jax.experimental import pallas as pl
from jax.experimental.pallas import tpu as pltpu

S = 4096
DIM = 2048
NH, HD = 16, 128
HH = HD // 2
BS = 8
TB = S // BS
SINK_B = 4
WIN_B = 4
CUR_BLOCK = TB - 1
WIN_START = CUR_BLOCK - WIN_B + 1
MB = 512 // BS
KL = SINK_B + WIN_B + MB
KSEL = KL * BS
SCALE = 1.0 / math.sqrt(HD)
NEG = -1e30
KOFF = NH * HD
VOFF = 2 * NH * HD


def _pair_swap(z):
    ev = jax.lax.broadcasted_iota(jnp.int32, z.shape, 1) % 2 == 0
    return jnp.where(ev, jnp.roll(z, -1, axis=1), jnp.roll(z, 1, axis=1))


def _qkvrope_kernel(x_ref, w_ref, ca_ref, sb_ref,
                    qr_ref, kr_ref, v_ref, kb_ref, ql_ref):
    i = pl.program_id(0)
    acc = jax.lax.dot_general(
        x_ref[:, :].astype(jnp.bfloat16), w_ref[:, :],
        (((1,), (0,)), ((), ())),
        preferred_element_type=jnp.float32)
    bm = acc.shape[0]
    ca = jnp.concatenate([ca_ref[:, :]] * NH, axis=1)
    sb = jnp.concatenate([sb_ref[:, :]] * NH, axis=1)
    q = acc[:, :KOFF]
    k = acc[:, KOFF:VOFF]
    qrot = q * ca + _pair_swap(q) * sb
    krot = k * ca + _pair_swap(k) * sb
    qr_ref[:, :] = qrot.astype(jnp.bfloat16)
    kr_ref[:, :] = krot.astype(jnp.bfloat16)
    kb_ref[:, :] = jnp.mean(krot.reshape(bm // BS, BS, KOFF), axis=1)
    v_ref[:, :] = acc[:, VOFF:].astype(jnp.bfloat16)

    @pl.when(i == pl.num_programs(0) - 1)
    def _():
        ql_ref[:, :] = acc[bm - 8:, :KOFF]


def _qkv_rope(x2, w_all, ca2, sb2, bm):
    return pl.pallas_call(
        _qkvrope_kernel,
        grid=(S // bm,),
        in_specs=[pl.BlockSpec((bm, DIM), lambda i: (i, 0)),
                  pl.BlockSpec((DIM, 3 * NH * HD), lambda i: (0, 0)),
                  pl.BlockSpec((bm, HD), lambda i: (i, 0)),
                  pl.BlockSpec((bm, HD), lambda i: (i, 0))],
        out_specs=[pl.BlockSpec((bm, NH * HD), lambda i: (i, 0)),
                   pl.BlockSpec((bm, NH * HD), lambda i: (i, 0)),
                   pl.BlockSpec((bm, NH * HD), lambda i: (i, 0)),
                   pl.BlockSpec((bm // BS, NH * HD), lambda i: (i, 0)),
                   pl.BlockSpec((8, NH * HD), lambda i: (0, 0))],
        out_shape=[jax.ShapeDtypeStruct((S, NH * HD), jnp.bfloat16),
                   jax.ShapeDtypeStruct((S, NH * HD), jnp.bfloat16),
                   jax.ShapeDtypeStruct((S, NH * HD), jnp.bfloat16),
                   jax.ShapeDtypeStruct((TB, NH * HD), jnp.float32),
                   jax.ShapeDtypeStruct((8, NH * HD), jnp.float32)],
    )(x2, w_all, ca2, sb2)


def _route_kernel(kb_ref, ql_ref, ca_ref, sb_ref, top_ref):
    ca = ca_ref[:, :]
    sb = sb_ref[:, :]
    qraw = ql_ref[7:8, :]
    cols = []
    for h in range(NH):
        qh = qraw[:, h * HD:(h + 1) * HD]
        qlh = qh * ca + _pair_swap(qh) * sb
        kb16 = kb_ref[:, h * HD:(h + 1) * HD].astype(jnp.bfloat16)
        ql16 = qlh.astype(jnp.bfloat16).astype(jnp.float32)
        cols.append(jnp.sum(kb16.astype(jnp.float32) * ql16,
                            axis=1, keepdims=True))
    scores = jnp.concatenate(cols, axis=1)
    rid = jax.lax.broadcasted_iota(jnp.int32, (TB, NH), 0)
    excl = (rid < SINK_B) | (rid >= WIN_START)
    scores = jnp.where(excl, NEG, scores)

    def body(j, sc):
        idx = jnp.argmax(sc, axis=0).astype(jnp.int32)
        top_ref[pl.ds(j, 1), :] = idx[None, :]
        hit = rid == idx[None, :]
        return jnp.where(hit, -jnp.inf, sc)

    jax.lax.fori_loop(0, MB, body, scores)


def _route(kb2, ql8, ca_last, sb_last):
    return pl.pallas_call(
        _route_kernel,
        grid=(1,),
        in_specs=[pl.BlockSpec((TB, NH * HD), lambda i: (0, 0)),
                  pl.BlockSpec((8, NH * HD), lambda i: (0, 0)),
                  pl.BlockSpec((1, HD), lambda i: (0, 0)),
                  pl.BlockSpec((1, HD), lambda i: (0, 0))],
        out_specs=pl.BlockSpec((MB, NH), lambda i: (0, 0)),
        out_shape=jax.ShapeDtypeStruct((MB, NH), jnp.int32),
    )(kb2, ql8, ca_last, sb_last)


def _attn_kernel(bi_ref, pos_ref, q_ref, k_ref, v_ref, o_ref, ks_ref, vs_ref):
    qt = pl.program_id(1)

    @pl.when(qt == 0)
    def _gather():
        def body(j, _):
            blk = bi_ref[0, 0, j]
            ks_ref[pl.ds(j * BS, BS), :] = k_ref[pl.ds(blk * BS, BS), :]
            vs_ref[pl.ds(j * BS, BS), :] = v_ref[pl.ds(blk * BS, BS), :]
            return 0
        jax.lax.fori_loop(0, KL, body, 0)

    q = q_ref[:, :]
    tq = q.shape[0]
    att = jax.lax.dot_general(
        q, ks_ref[:, :], (((1,), (1,)), ((), ())),
        preferred_element_type=jnp.float32) * SCALE
    qpos = (qt * tq + jax.lax.broadcasted_iota(jnp.int32, (tq, 1), 0)
            ).astype(jnp.float32)
    allow = pos_ref[0, :, :] <= qpos
    att = jnp.where(allow, att, NEG)
    m = jnp.max(att, axis=1, keepdims=True)
    e = jnp.exp(att - m)
    denom = jnp.sum(e, axis=1, keepdims=True)
    y = jax.lax.dot_general(
        e.astype(jnp.bfloat16), vs_ref[:, :], (((1,), (0,)), ((), ())),
        preferred_element_type=jnp.float32)
    o_ref[:, :] = (y / denom).astype(o_ref.dtype)


def _attention(block_index, pos3, q2, k2, v2, tq):
    return pl.pallas_call(
        _attn_kernel,
        grid=(NH, S // tq),
        in_specs=[
            pl.BlockSpec((1, 1, KL), lambda h, i: (h, 0, 0),
                         memory_space=pltpu.SMEM),
            pl.BlockSpec((1, 1, KSEL), lambda h, i: (h, 0, 0)),
            pl.BlockSpec((tq, HD), lambda h, i: (i, h)),
            pl.BlockSpec((S, HD), lambda h, i: (0, h)),
            pl.BlockSpec((S, HD), lambda h, i: (0, h)),
        ],
        out_specs=pl.BlockSpec((tq, HD), lambda h, i: (i, h)),
        out_shape=jax.ShapeDtypeStruct((S, NH * HD), jnp.bfloat16),
        scratch_shapes=[pltpu.VMEM((KSEL, HD), jnp.bfloat16),
                        pltpu.VMEM((KSEL, HD), jnp.bfloat16)],
    )(block_index.reshape(NH, 1, KL), pos3, q2, k2, v2)


def _mm_kernel(a_ref, b_ref, o_ref):
    o_ref[:, :] = jax.lax.dot_general(
        a_ref[:, :], b_ref[:, :].astype(jnp.bfloat16),
        (((1,), (0,)), ((), ())),
        preferred_element_type=jnp.float32)


def _out_proj(a, b, bm):
    m, k = a.shape
    k2, n = b.shape
    return pl.pallas_call(
        _mm_kernel,
        grid=(m // bm,),
        in_specs=[pl.BlockSpec((bm, k), lambda i: (i, 0)),
                  pl.BlockSpec((k, n), lambda i: (0, 0))],
        out_specs=pl.BlockSpec((bm, n), lambda i: (i, 0)),
        out_shape=jax.ShapeDtypeStruct((m, n), jnp.float32),
    )(a, b)


def kernel(x, freqs_cis, wqkv, wo, input_pos):
    x2 = x[0]
    w_all = wqkv.astype(jnp.bfloat16)

    c = freqs_cis[:, :, 0]
    s = freqs_cis[:, :, 1]
    ca2 = jnp.repeat(c, 2, axis=1)
    sb2 = jnp.stack([-s, s], axis=-1).reshape(S, HD)

    q_rot, k_rot, v2, kb2, ql8 = _qkv_rope(x2, w_all, ca2, sb2, 256)

    ca_last = ca2[S - 1].reshape(1, HD)
    sb_last = sb2[S - 1].reshape(1, HD)
    top = _route(kb2, ql8, ca_last, sb_last)

    fixed = jnp.concatenate([
        jnp.arange(SINK_B, dtype=jnp.int32),
        jnp.arange(WIN_START, CUR_BLOCK + 1, dtype=jnp.int32)])
    block_index = jnp.concatenate(
        [jnp.broadcast_to(fixed[None, :], (NH, SINK_B + WIN_B)),
         top.T], axis=1)
    pos3 = (block_index[:, :, None] * BS
            + jnp.arange(BS, dtype=jnp.int32)[None, None, :]
            ).reshape(NH, 1, KSEL).astype(jnp.float32)

    y2 = _attention(block_index, pos3, q_rot, k_rot, v2, 256)

    out = _out_proj(y2, wo, 512)
    return out.reshape(1, S, DIM)


# speedup vs baseline: 3.1461x; 1.1286x over previous
import math

import jax
import jax.numpy as jnp
from jax.experimental import pallas as pl
from jax.experimental.pallas import tpu as pltpu

S = 4096
DIM = 2048
NH, HD = 16, 128
HH = HD // 2
BS = 8
TB = S // BS
SINK_B = 4
WIN_B = 4
CUR_BLOCK = TB - 1
WIN_START = CUR_BLOCK - WIN_B + 1
MB = 512 // BS
KL = SINK_B + WIN_B + MB
KSEL = KL * BS
SCALE = 1.0 / math.sqrt(HD)
NEG = -1e30
KOFF = NH * HD
VOFF = 2 * NH * HD


def _pair_swap(z):
    ev = jax.lax.broadcasted_iota(jnp.int32, z.shape, 1) % 2 == 0
    return jnp.where(ev, jnp.roll(z, -1, axis=1), jnp.roll(z, 1, axis=1))


def _qkvrope_kernel(x_ref, w_ref, ca_ref, sb_ref,
                    qr_ref, kr_ref, v_ref, kb_ref, ql_ref):
    i = pl.program_id(0)
    acc = jax.lax.dot_general(
        x_ref[:, :].astype(jnp.bfloat16), w_ref[:, :],
        (((1,), (0,)), ((), ())),
        preferred_element_type=jnp.float32)
    bm = acc.shape[0]
    ca = jnp.concatenate([ca_ref[:, :]] * NH, axis=1)
    sb = jnp.concatenate([sb_ref[:, :]] * NH, axis=1)
    q = acc[:, :KOFF]
    k = acc[:, KOFF:VOFF]
    qrot = q * ca + _pair_swap(q) * sb
    krot = k * ca + _pair_swap(k) * sb
    qr_ref[:, :] = qrot.astype(jnp.bfloat16)
    kr_ref[:, :] = krot.astype(jnp.bfloat16)
    kb_ref[:, :] = jnp.mean(krot.reshape(bm // BS, BS, KOFF), axis=1)
    v_ref[:, :] = acc[:, VOFF:].astype(jnp.bfloat16)

    @pl.when(i == pl.num_programs(0) - 1)
    def _():
        ql_ref[:, :] = acc[bm - 8:, :KOFF]


def _qkv_rope(x2, w_all, ca2, sb2, bm):
    return pl.pallas_call(
        _qkvrope_kernel,
        grid=(S // bm,),
        in_specs=[pl.BlockSpec((bm, DIM), lambda i: (i, 0)),
                  pl.BlockSpec((DIM, 3 * NH * HD), lambda i: (0, 0)),
                  pl.BlockSpec((bm, HD), lambda i: (i, 0)),
                  pl.BlockSpec((bm, HD), lambda i: (i, 0))],
        out_specs=[pl.BlockSpec((bm, NH * HD), lambda i: (i, 0)),
                   pl.BlockSpec((bm, NH * HD), lambda i: (i, 0)),
                   pl.BlockSpec((bm, NH * HD), lambda i: (i, 0)),
                   pl.BlockSpec((bm // BS, NH * HD), lambda i: (i, 0)),
                   pl.BlockSpec((8, NH * HD), lambda i: (0, 0))],
        out_shape=[jax.ShapeDtypeStruct((S, NH * HD), jnp.bfloat16),
                   jax.ShapeDtypeStruct((S, NH * HD), jnp.bfloat16),
                   jax.ShapeDtypeStruct((S, NH * HD), jnp.bfloat16),
                   jax.ShapeDtypeStruct((TB, NH * HD), jnp.float32),
                   jax.ShapeDtypeStruct((8, NH * HD), jnp.float32)],
    )(x2, w_all, ca2, sb2)


def _route_kernel(kb_ref, ql_ref, ca_ref, sb_ref, top_ref):
    ca = ca_ref[:, :]
    sb = sb_ref[:, :]
    qraw = ql_ref[7:8, :]
    cols = []
    for h in range(NH):
        qh = qraw[:, h * HD:(h + 1) * HD]
        qlh = qh * ca + _pair_swap(qh) * sb
        kb16 = kb_ref[:, h * HD:(h + 1) * HD].astype(jnp.bfloat16)
        ql16 = qlh.astype(jnp.bfloat16).astype(jnp.float32)
        cols.append(jnp.sum(kb16.astype(jnp.float32) * ql16,
                            axis=1, keepdims=True))
    scores = jnp.concatenate(cols, axis=1)
    rid = jax.lax.broadcasted_iota(jnp.int32, (TB, NH), 0)
    excl = (rid < SINK_B) | (rid >= WIN_START)
    scores = jnp.where(excl, NEG, scores)

    def body(j, sc):
        idx = jnp.argmax(sc, axis=0).astype(jnp.int32)
        top_ref[pl.ds(j, 1), :] = idx[None, :]
        hit = rid == idx[None, :]
        return jnp.where(hit, -jnp.inf, sc)

    jax.lax.fori_loop(0, MB, body, scores)


def _route(kb2, ql8, ca_last, sb_last):
    return pl.pallas_call(
        _route_kernel,
        grid=(1,),
        in_specs=[pl.BlockSpec((TB, NH * HD), lambda i: (0, 0)),
                  pl.BlockSpec((8, NH * HD), lambda i: (0, 0)),
                  pl.BlockSpec((1, HD), lambda i: (0, 0)),
                  pl.BlockSpec((1, HD), lambda i: (0, 0))],
        out_specs=pl.BlockSpec((MB, NH), lambda i: (0, 0)),
        out_shape=jax.ShapeDtypeStruct((MB, NH), jnp.int32),
    )(kb2, ql8, ca_last, sb_last)


def _attn_kernel(bi_ref, pos_ref, q_ref, k_ref, v_ref, o_ref, ks_ref, vs_ref):
    qt = pl.program_id(1)

    @pl.when(qt == 0)
    def _gather():
        def body(j, _):
            blk = bi_ref[0, 0, j]
            ks_ref[pl.ds(j * BS, BS), :] = k_ref[pl.ds(blk * BS, BS), :]
            vs_ref[pl.ds(j * BS, BS), :] = v_ref[pl.ds(blk * BS, BS), :]
            return 0
        jax.lax.fori_loop(0, KL, body, 0)

    q = q_ref[:, :]
    tq = q.shape[0]
    att = jax.lax.dot_general(
        q, ks_ref[:, :], (((1,), (1,)), ((), ())),
        preferred_element_type=jnp.float32) * SCALE
    qpos = (qt * tq + jax.lax.broadcasted_iota(jnp.int32, (tq, 1), 0)
            ).astype(jnp.float32)
    allow = pos_ref[0, :, :] <= qpos
    att = jnp.where(allow, att, NEG)
    m = jnp.max(att, axis=1, keepdims=True)
    e = jnp.exp(att - m)
    denom = jnp.sum(e, axis=1, keepdims=True)
    y = jax.lax.dot_general(
        e.astype(jnp.bfloat16), vs_ref[:, :], (((1,), (0,)), ((), ())),
        preferred_element_type=jnp.float32)
    o_ref[:, :] = (y / denom).astype(o_ref.dtype)


def _attention(block_index, pos3, q2, k2, v2, tq):
    return pl.pallas_call(
        _attn_kernel,
        grid=(NH, S // tq),
        in_specs=[
            pl.BlockSpec((1, 1, KL), lambda h, i: (h, 0, 0),
                         memory_space=pltpu.SMEM),
            pl.BlockSpec((1, 1, KSEL), lambda h, i: (h, 0, 0)),
            pl.BlockSpec((tq, HD), lambda h, i: (i, h)),
            pl.BlockSpec((S, HD), lambda h, i: (0, h)),
            pl.BlockSpec((S, HD), lambda h, i: (0, h)),
        ],
        out_specs=pl.BlockSpec((tq, HD), lambda h, i: (i, h)),
        out_shape=jax.ShapeDtypeStruct((S, NH * HD), jnp.bfloat16),
        scratch_shapes=[pltpu.VMEM((KSEL, HD), jnp.bfloat16),
                        pltpu.VMEM((KSEL, HD), jnp.bfloat16)],
    )(block_index.reshape(NH, 1, KL), pos3, q2, k2, v2)


def _mm_kernel(a_ref, b_ref, o_ref):
    o_ref[:, :] = jax.lax.dot_general(
        a_ref[:, :], b_ref[:, :], (((1,), (0,)), ((), ())),
        preferred_element_type=jnp.float32)


def _out_proj(a, b, bm):
    m, k = a.shape
    k2, n = b.shape
    return pl.pallas_call(
        _mm_kernel,
        grid=(m // bm,),
        in_specs=[pl.BlockSpec((bm, k), lambda i: (i, 0)),
                  pl.BlockSpec((k, n), lambda i: (0, 0))],
        out_specs=pl.BlockSpec((bm, n), lambda i: (i, 0)),
        out_shape=jax.ShapeDtypeStruct((m, n), jnp.float32),
    )(a, b)


def kernel(x, freqs_cis, wqkv, wo, input_pos):
    x2 = x[0]
    w_all = wqkv.astype(jnp.bfloat16)

    c = freqs_cis[:, :, 0]
    s = freqs_cis[:, :, 1]
    ca2 = jnp.repeat(c, 2, axis=1)
    sb2 = jnp.stack([-s, s], axis=-1).reshape(S, HD)

    q_rot, k_rot, v2, kb2, ql8 = _qkv_rope(x2, w_all, ca2, sb2, 256)

    ca_last = ca2[S - 1].reshape(1, HD)
    sb_last = sb2[S - 1].reshape(1, HD)
    top = _route(kb2, ql8, ca_last, sb_last)

    fixed = jnp.concatenate([
        jnp.arange(SINK_B, dtype=jnp.int32),
        jnp.arange(WIN_START, CUR_BLOCK + 1, dtype=jnp.int32)])
    block_index = jnp.concatenate(
        [jnp.broadcast_to(fixed[None, :], (NH, SINK_B + WIN_B)),
         top.T], axis=1)
    pos3 = (block_index[:, :, None] * BS
            + jnp.arange(BS, dtype=jnp.int32)[None, None, :]
            ).reshape(NH, 1, KSEL).astype(jnp.float32)

    y2 = _attention(block_index, pos3, q_rot, k_rot, v2, 512)

    out = _out_proj(y2, wo.astype(jnp.bfloat16), 512)
    return out.reshape(1, S, DIM)
